# baseline (speedup 1.0000x reference)
import jax
import jax.numpy as jnp
from jax.experimental import pallas as pl
from jax.experimental.pallas import tpu as pltpu


def _up_kernel(x_ref, w_ref, b_ref, o_ref):
    H, two, W, C2 = o_ref.shape
    y = jax.lax.dot_general(x_ref[...], w_ref[...], (((0,), (0,)), ((), ())),
                            preferred_element_type=jnp.float32) + b_ref[...]
    for di in range(2):
        o_ref[:, di:di + 1, :, :] = (
            y[:, di * C2:(di + 1) * C2].reshape(H, 1, W, C2).astype(o_ref.dtype))


def _upsample(x_nchw, w_t, b):
    N, Cin, H, W = x_nchw.shape
    Cout = w_t.shape[1]
    C2 = 2 * Cout
    w4 = jnp.transpose(w_t, (0, 2, 3, 1)).reshape(Cin, 4 * Cout).astype(jnp.bfloat16)
    b4 = jnp.tile(b, 4).reshape(1, 4 * Cout)
    x_cm = x_nchw.reshape(N, Cin, H * W).astype(jnp.bfloat16)

    y = pl.pallas_call(
        _up_kernel,
        out_shape=jax.ShapeDtypeStruct((N, H, 2, W, C2), jnp.bfloat16),
        grid=(N,),
        in_specs=[
            pl.BlockSpec((None, Cin, H * W), lambda n: (n, 0, 0)),
            pl.BlockSpec((Cin, 4 * Cout), lambda n: (0, 0)),
            pl.BlockSpec((1, 4 * Cout), lambda n: (0, 0)),
        ],
        out_specs=pl.BlockSpec((None, H, 2, W, C2), lambda n: (n, 0, 0, 0, 0)),
        compiler_params=pltpu.CompilerParams(dimension_semantics=("parallel",)),
    )(x_cm, w4, b4)
    return y.reshape(N, 2 * H, 2 * W, Cout)


def _res2_kernel(up_ref, br_ref, w1a_ref, b1a_ref, w2a_ref, b2a_ref,
                 w1b_ref, b1b_ref, w2b_ref, b2b_ref, o_ref,
                 apad_ref, zcat_ref):
    H2, W2, C = o_ref.shape
    Wp = W2 + 2
    apad_ref[...] = jnp.zeros_like(apad_ref)

    def write_z(a):
        apad_ref[1:H2 + 1, 1:W2 + 1, :] = jnp.where(a >= 0, a, 0.2 * a)
        zcat_ref[...] = apad_ref[...].astype(jnp.bfloat16)

    def conv3x3(w_ref, b_ref):
        strips = []
        for s in range(0, H2, 16):
            lhs = jnp.concatenate(
                [zcat_ref[s + ky:s + ky + 16].reshape(16 * Wp, C)
                 for ky in range(3)], axis=1)
            u = jnp.dot(lhs, w_ref[...],
                        preferred_element_type=jnp.float32).reshape(16, Wp, 3 * C)
            strips.append(u[:, 0:W2, 0:C] + u[:, 1:W2 + 1, C:2 * C]
                          + u[:, 2:W2 + 2, 2 * C:3 * C] + b_ref[...])
        return jnp.concatenate(strips, axis=0)

    o_ref[...] = up_ref[...].astype(jnp.float32) + br_ref[...].astype(jnp.float32)
    write_z(o_ref[...])
    write_z(conv3x3(w1a_ref, b1a_ref))
    o_ref[...] = o_ref[...] + conv3x3(w2a_ref, b2a_ref)
    write_z(o_ref[...])
    write_z(conv3x3(w1b_ref, b1b_ref))
    o_ref[...] = o_ref[...] + conv3x3(w2b_ref, b2b_ref)


def _res2(up_nhwc, br_nhwc, w1a, b1a, w2a, b2a, w1b, b1b, w2b, b2b):
    N, H2, W2, C = up_nhwc.shape
    wspec = pl.BlockSpec((3 * C, 3 * C), lambda n: (0, 0))
    bspec = pl.BlockSpec((1, C), lambda n: (0, 0))
    bf = jnp.bfloat16

    def wk(w):
        return jnp.transpose(w, (0, 2, 1, 3)).reshape(3 * C, 3 * C).astype(bf)

    bfspec = pl.BlockSpec((None, H2, W2, C), lambda n: (n, 0, 0, 0))
    return pl.pallas_call(
        _res2_kernel,
        out_shape=jax.ShapeDtypeStruct((N, H2, W2, C), jnp.float32),
        grid=(N,),
        in_specs=[bfspec, bfspec,
                  wspec, bspec, wspec, bspec, wspec, bspec, wspec, bspec],
        out_specs=bfspec,
        scratch_shapes=[pltpu.VMEM((H2 + 2, W2 + 2, C), jnp.float32),
                        pltpu.VMEM((H2 + 2, W2 + 2, C), bf)],
        compiler_params=pltpu.CompilerParams(dimension_semantics=("parallel",)),
    )(up_nhwc, br_nhwc,
      wk(w1a), b1a.reshape(1, C), wk(w2a), b2a.reshape(1, C),
      wk(w1b), b1b.reshape(1, C), wk(w2b), b2b.reshape(1, C))


def kernel(x_nchw, bridge_nchw, up_w, up_b, w1_0, b1_0, w2_0, b2_0,
           w1_1, b1_1, w2_1, b2_1):
    up = _upsample(x_nchw, up_w, up_b)
    br = jnp.transpose(bridge_nchw, (0, 2, 3, 1)).astype(jnp.bfloat16)
    out = _res2(up, br, w1_0, b1_0, w2_0, b2_0, w1_1, b1_1, w2_1, b2_1)
    return jnp.transpose(out, (0, 3, 1, 2))


# speedup vs baseline: 2.0687x; 2.0687x over previous
import jax
import jax.numpy as jnp
from jax.experimental import pallas as pl
from jax.experimental.pallas import tpu as pltpu


def _up_kernel(x_ref, w_ref, b_ref, o_ref):
    H, two, W, C2 = o_ref.shape
    y = jax.lax.dot_general(x_ref[...], w_ref[...], (((0,), (0,)), ((), ())),
                            preferred_element_type=jnp.float32) + b_ref[...]
    for di in range(2):
        o_ref[:, di:di + 1, :, :] = (
            y[:, di * C2:(di + 1) * C2].reshape(H, 1, W, C2).astype(o_ref.dtype))


def _upsample(x_nchw, w_t, b):
    N, Cin, H, W = x_nchw.shape
    Cout = w_t.shape[1]
    C2 = 2 * Cout
    w4 = jnp.transpose(w_t, (0, 2, 3, 1)).reshape(Cin, 4 * Cout).astype(jnp.bfloat16)
    b4 = jnp.tile(b, 4).reshape(1, 4 * Cout)
    x_cm = x_nchw.reshape(N, Cin, H * W).astype(jnp.bfloat16)

    y = pl.pallas_call(
        _up_kernel,
        out_shape=jax.ShapeDtypeStruct((N, H, 2, W, C2), jnp.bfloat16),
        grid=(N,),
        in_specs=[
            pl.BlockSpec((None, Cin, H * W), lambda n: (n, 0, 0)),
            pl.BlockSpec((Cin, 4 * Cout), lambda n: (0, 0)),
            pl.BlockSpec((1, 4 * Cout), lambda n: (0, 0)),
        ],
        out_specs=pl.BlockSpec((None, H, 2, W, C2), lambda n: (n, 0, 0, 0, 0)),
        compiler_params=pltpu.CompilerParams(dimension_semantics=("parallel",)),
    )(x_cm, w4, b4)
    return y.reshape(N, 2 * H, 2 * W, Cout)


def _res2_kernel(up_ref, br_ref, w1a_ref, b1a_ref, w2a_ref, b2a_ref,
                 w1b_ref, b1b_ref, w2b_ref, b2b_ref, o_ref,
                 apad_ref, zcat_ref):
    H2, W2, C = o_ref.shape
    apad_ref[...] = jnp.zeros_like(apad_ref)

    def write_z(a):
        apad_ref[1:H2 + 1, 1:W2 + 1, :] = jnp.where(a >= 0, a, 0.2 * a)
        for kx in range(3):
            zcat_ref[:, :, kx * C:(kx + 1) * C] = (
                apad_ref[:, kx:kx + W2, :].astype(jnp.bfloat16))

    def conv3x3(w_ref, b_ref):
        lhs01 = jnp.concatenate(
            [zcat_ref[0:H2].reshape(H2 * W2, 3 * C),
             zcat_ref[1:H2 + 1].reshape(H2 * W2, 3 * C)], axis=1)
        acc = jnp.dot(lhs01, w_ref[0:6 * C, :],
                      preferred_element_type=jnp.float32)
        acc = acc + jnp.dot(zcat_ref[2:H2 + 2].reshape(H2 * W2, 3 * C),
                            w_ref[6 * C:, :], preferred_element_type=jnp.float32)
        return (acc + b_ref[...]).reshape(H2, W2, C)

    o_ref[...] = up_ref[...].astype(jnp.float32) + br_ref[...].astype(jnp.float32)
    write_z(o_ref[...])
    write_z(conv3x3(w1a_ref, b1a_ref))
    o_ref[...] = o_ref[...] + conv3x3(w2a_ref, b2a_ref)
    write_z(o_ref[...])
    write_z(conv3x3(w1b_ref, b1b_ref))
    o_ref[...] = o_ref[...] + conv3x3(w2b_ref, b2b_ref)


def _res2(up_nhwc, br_nhwc, w1a, b1a, w2a, b2a, w1b, b1b, w2b, b2b):
    N, H2, W2, C = up_nhwc.shape
    wspec = pl.BlockSpec((9 * C, C), lambda n: (0, 0))
    bspec = pl.BlockSpec((1, C), lambda n: (0, 0))
    bf = jnp.bfloat16

    def wk(w):
        return w.reshape(9 * C, C).astype(bf)

    bfspec = pl.BlockSpec((None, H2, W2, C), lambda n: (n, 0, 0, 0))
    return pl.pallas_call(
        _res2_kernel,
        out_shape=jax.ShapeDtypeStruct((N, H2, W2, C), jnp.float32),
        grid=(N,),
        in_specs=[bfspec, bfspec,
                  wspec, bspec, wspec, bspec, wspec, bspec, wspec, bspec],
        out_specs=bfspec,
        scratch_shapes=[pltpu.VMEM((H2 + 2, W2 + 2, C), jnp.float32),
                        pltpu.VMEM((H2 + 2, W2, 3 * C), bf)],
        compiler_params=pltpu.CompilerParams(dimension_semantics=("parallel",)),
    )(up_nhwc, br_nhwc,
      wk(w1a), b1a.reshape(1, C), wk(w2a), b2a.reshape(1, C),
      wk(w1b), b1b.reshape(1, C), wk(w2b), b2b.reshape(1, C))


def kernel(x_nchw, bridge_nchw, up_w, up_b, w1_0, b1_0, w2_0, b2_0,
           w1_1, b1_1, w2_1, b2_1):
    up = _upsample(x_nchw, up_w, up_b)
    br = jnp.transpose(bridge_nchw, (0, 2, 3, 1)).astype(jnp.bfloat16)
    out = _res2(up, br, w1_0, b1_0, w2_0, b2_0, w1_1, b1_1, w2_1, b2_1)
    return jnp.transpose(out, (0, 3, 1, 2))
